# baseline (speedup 1.0000x reference)
import functools

import jax
import jax.numpy as jnp
from jax import lax
from jax.experimental import pallas as pl
from jax.experimental.pallas import tpu as pltpu


def _fused_kernel(
    af_ref,
    z1_ref,
    wm_ref,
    bm_ref,
    gw_ref,
    gb_ref,
    w1_ref,
    w2_ref,
    b12_ref,
    w3_ref,
    b3_ref,
    o_ref,
    *,
    t_len,
    n_chunks,
):
    R = z1_ref.shape[0]
    cin = z1_ref.shape[1]
    cout = o_ref.shape[1]
    rc = R // n_chunks
    nbc = rc // t_len

    t_idx = lax.broadcasted_iota(jnp.int32, (rc, 1), 0) % t_len
    is_first = t_idx == 0
    is_last = t_idx == (t_len - 1)

    a_b = jnp.dot(af_ref[...], wm_ref[...],
                  preferred_element_type=jnp.float32) + bm_ref[...]
    a_b16 = a_b.astype(jnp.bfloat16)

    for c in range(n_chunks):
        rows = pl.ds(c * rc, rc)

        ab_c = a_b16[c * nbc:(c + 1) * nbc]
        a = jnp.broadcast_to(ab_c[:, None, :], (nbc, t_len, cin)
                             ).reshape(rc, cin)
        af_glow = gw_ref[rows] * a + gb_ref[rows]

        z = jnp.concatenate([z1_ref[rows, :].astype(jnp.bfloat16), af_glow],
                            axis=-1)
        z_prev = jnp.where(is_first, 0, pltpu.roll(z, 1, axis=0))
        z_next = jnp.where(is_last, 0, pltpu.roll(z, rc - 1, axis=0))
        zs = jnp.concatenate([z_prev, z, z_next], axis=-1)
        y = jnp.dot(zs, w1_ref[...], preferred_element_type=jnp.float32)
        y = jnp.maximum(y.astype(jnp.bfloat16) + b12_ref[0:1, :], 0)

        y = jnp.dot(y, w2_ref[...], preferred_element_type=jnp.float32)
        y = jnp.maximum(y.astype(jnp.bfloat16) + b12_ref[1:2, :], 0)

        p = jnp.dot(y, w3_ref[...], preferred_element_type=jnp.float32)
        p_prev = pltpu.roll(p[:, :cout], 1, axis=0)
        p_next = pltpu.roll(p[:, 2 * cout:], rc - 1, axis=0)
        out = (p[:, cout:2 * cout]
               + jnp.where(is_first, 0.0, p_prev)
               + jnp.where(is_last, 0.0, p_next)
               + b3_ref[...])
        o_ref[rows, :] = out


@jax.jit
def kernel(z1, audio_features, w_mlp_t, b_mlp, glow, w1s, w2m, b12, w3s, b3):
    N, T, Cin = z1.shape
    Dc = audio_features.shape[1]
    H = w2m.shape[0]
    Cout = b3.shape[1]

    max_rows = 8192
    nb = N
    if N * T > max_rows:
        for cand in range(min(N, max(1, max_rows // T)), 0, -1):
            if N % cand == 0 and cand % 8 == 0:
                nb = cand
                break
    grid = (N // nb,)
    R = nb * T
    n_chunks = 32
    while nb % n_chunks != 0:
        n_chunks //= 2

    z1_flat = z1.reshape(N * T, Cin)
    af_b = audio_features.astype(jnp.bfloat16)
    wm_b = w_mlp_t.astype(jnp.bfloat16)
    w1_b = w1s.astype(jnp.bfloat16)
    w2_b = w2m.astype(jnp.bfloat16)
    w3_b = jnp.concatenate([w3s[0:H], w3s[H:2 * H], w3s[2 * H:3 * H]],
                           axis=1).astype(jnp.bfloat16)
    gw = jnp.tile(glow[:, 0:1], (nb, 1)).astype(jnp.bfloat16)
    gb = jnp.tile(glow[:, 1:2], (nb, 1)).astype(jnp.bfloat16)
    b12_b = b12.astype(jnp.bfloat16)

    plist = [wm_b, b_mlp, gw, gb, w1_b, w2_b, b12_b, w3_b, b3]

    in_specs = [
        pl.BlockSpec((nb, Dc), lambda g: (g, 0)),
        pl.BlockSpec((R, Cin), lambda g: (g, 0)),
    ] + [pl.BlockSpec(p.shape, lambda g: (0, 0)) for p in plist]

    out = pl.pallas_call(
        functools.partial(_fused_kernel, t_len=T, n_chunks=n_chunks),
        out_shape=jax.ShapeDtypeStruct((N * T, Cout), jnp.float32),
        grid=grid,
        in_specs=in_specs,
        out_specs=pl.BlockSpec((R, Cout), lambda g: (g, 0)),
        compiler_params=pltpu.CompilerParams(
            dimension_semantics=("parallel",),
            vmem_limit_bytes=64 * 1024 * 1024),
    )(af_b, z1_flat, *plist)
    return out.reshape(N, T, Cout)


# speedup vs baseline: 1.1070x; 1.1070x over previous
import functools

import jax
import jax.numpy as jnp
from jax import lax
from jax.experimental import pallas as pl
from jax.experimental.pallas import tpu as pltpu


def _fused_kernel(
    af_ref,
    z1_ref,
    wm_ref,
    bm_ref,
    gw_ref,
    gb_ref,
    w1_ref,
    w2_ref,
    b12_ref,
    w3_ref,
    b3_ref,
    o_ref,
    *,
    t_len,
    n_chunks,
):
    R = z1_ref.shape[0]
    cin = z1_ref.shape[1]
    cout = o_ref.shape[1]
    rc = R // n_chunks
    nbc = rc // t_len

    t_idx = lax.broadcasted_iota(jnp.int32, (rc, 1), 0) % t_len
    is_first = t_idx == 0
    is_last = t_idx == (t_len - 1)

    a_b = jnp.dot(af_ref[...], wm_ref[...],
                  preferred_element_type=jnp.float32) + bm_ref[...]
    a_b16 = a_b.astype(jnp.bfloat16)

    for c in range(n_chunks):
        rows = pl.ds(c * rc, rc)

        ab_c = a_b16[c * nbc:(c + 1) * nbc]
        a = jnp.broadcast_to(ab_c[:, None, :], (nbc, t_len, cin)
                             ).reshape(rc, cin)
        af_glow = gw_ref[rows] * a + gb_ref[rows]

        z = jnp.concatenate([z1_ref[rows, :].astype(jnp.bfloat16), af_glow],
                            axis=-1)
        z_prev = jnp.where(is_first, 0, pltpu.roll(z, 1, axis=0))
        z_next = jnp.where(is_last, 0, pltpu.roll(z, rc - 1, axis=0))
        zs = jnp.concatenate([z_prev, z, z_next], axis=-1)
        y = jnp.dot(zs, w1_ref[...], preferred_element_type=jnp.float32)
        y = jnp.maximum(y.astype(jnp.bfloat16) + b12_ref[0:1, :], 0)

        y = jnp.dot(y, w2_ref[...], preferred_element_type=jnp.float32)
        y = jnp.maximum(y.astype(jnp.bfloat16) + b12_ref[1:2, :], 0)

        p = jnp.dot(y, w3_ref[...], preferred_element_type=jnp.float32)
        p_prev = pltpu.roll(p[:, :cout], 1, axis=0)
        p_next = pltpu.roll(p[:, 2 * cout:], rc - 1, axis=0)
        out = (p[:, cout:2 * cout]
               + jnp.where(is_first, 0.0, p_prev)
               + jnp.where(is_last, 0.0, p_next)
               + b3_ref[...])
        o_ref[rows, :] = out


@jax.jit
def kernel(z1, audio_features, w_mlp_t, b_mlp, glow, w1s, w2m, b12, w3s, b3):
    N, T, Cin = z1.shape
    Dc = audio_features.shape[1]
    H = w2m.shape[0]
    Cout = b3.shape[1]

    max_rows = 16384
    nb = N
    if N * T > max_rows:
        for cand in range(min(N, max(1, max_rows // T)), 0, -1):
            if N % cand == 0 and cand % 8 == 0:
                nb = cand
                break
    grid = (N // nb,)
    R = nb * T
    n_chunks = max(1, R // 512)
    while nb % n_chunks != 0:
        n_chunks //= 2

    z1_flat = z1.reshape(N * T, Cin)
    af_b = audio_features.astype(jnp.bfloat16)
    wm_b = w_mlp_t.astype(jnp.bfloat16)
    w1_b = w1s.astype(jnp.bfloat16)
    w2_b = w2m.astype(jnp.bfloat16)
    w3_b = jnp.concatenate([w3s[0:H], w3s[H:2 * H], w3s[2 * H:3 * H]],
                           axis=1).astype(jnp.bfloat16)
    gw = jnp.tile(glow[:, 0:1], (nb, 1)).astype(jnp.bfloat16)
    gb = jnp.tile(glow[:, 1:2], (nb, 1)).astype(jnp.bfloat16)
    b12_b = b12.astype(jnp.bfloat16)

    plist = [wm_b, b_mlp, gw, gb, w1_b, w2_b, b12_b, w3_b, b3]

    in_specs = [
        pl.BlockSpec((nb, Dc), lambda g: (g, 0)),
        pl.BlockSpec((R, Cin), lambda g: (g, 0)),
    ] + [pl.BlockSpec(p.shape, lambda g: (0, 0)) for p in plist]

    out = pl.pallas_call(
        functools.partial(_fused_kernel, t_len=T, n_chunks=n_chunks),
        out_shape=jax.ShapeDtypeStruct((N * T, Cout), jnp.float32),
        grid=grid,
        in_specs=in_specs,
        out_specs=pl.BlockSpec((R, Cout), lambda g: (g, 0)),
        compiler_params=pltpu.CompilerParams(
            dimension_semantics=("parallel",),
            vmem_limit_bytes=64 * 1024 * 1024),
    )(af_b, z1_flat, *plist)
    return out.reshape(N, T, Cout)


# speedup vs baseline: 1.1640x; 1.0516x over previous
import functools

import jax
import jax.numpy as jnp
from jax import lax
from jax.experimental import pallas as pl
from jax.experimental.pallas import tpu as pltpu


def _fused_kernel(
    af_ref,
    z1_ref,
    wm_ref,
    bm_ref,
    gw_ref,
    gb_ref,
    w1_ref,
    w2_ref,
    b12_ref,
    w3_ref,
    b3_ref,
    o_ref,
    *,
    t_len,
    n_chunks,
):
    R = z1_ref.shape[0]
    cin = z1_ref.shape[1]
    cout = o_ref.shape[1]
    rc = R // n_chunks
    nbc = rc // t_len

    t_idx = lax.broadcasted_iota(jnp.int32, (rc, 1), 0) % t_len
    is_first = t_idx == 0
    is_last = t_idx == (t_len - 1)

    a_b = jnp.dot(af_ref[...], wm_ref[...],
                  preferred_element_type=jnp.float32) + bm_ref[...]
    a_b16 = a_b.astype(jnp.bfloat16)

    for c in range(n_chunks):
        rows = pl.ds(c * rc, rc)

        ab_c = a_b16[c * nbc:(c + 1) * nbc]
        a = jnp.broadcast_to(ab_c[:, None, :], (nbc, t_len, cin)
                             ).reshape(rc, cin)
        af_glow = gw_ref[rows] * a + gb_ref[rows]

        z = jnp.concatenate([z1_ref[rows, :].astype(jnp.bfloat16), af_glow],
                            axis=-1)
        z_prev = jnp.where(is_first, 0, pltpu.roll(z, 1, axis=0))
        z_next = jnp.where(is_last, 0, pltpu.roll(z, rc - 1, axis=0))
        zs = jnp.concatenate([z_prev, z, z_next], axis=-1)
        y = jnp.dot(zs, w1_ref[...], preferred_element_type=jnp.float32)
        y = jnp.maximum(y.astype(jnp.bfloat16) + b12_ref[0:1, :], 0)

        y = jnp.dot(y, w2_ref[...], preferred_element_type=jnp.float32)
        y = jnp.maximum(y.astype(jnp.bfloat16) + b12_ref[1:2, :], 0)

        p = jnp.dot(y, w3_ref[...], preferred_element_type=jnp.float32)
        p_prev = pltpu.roll(p[:, :cout], 1, axis=0)
        p_next = pltpu.roll(p[:, 2 * cout:], rc - 1, axis=0)
        out = (p[:, cout:2 * cout]
               + jnp.where(is_first, 0.0, p_prev)
               + jnp.where(is_last, 0.0, p_next)
               + b3_ref[...])
        o_ref[rows, :] = out


@jax.jit
def kernel(z1, audio_features, w_mlp_t, b_mlp, glow, w1s, w2m, b12, w3s, b3):
    N, T, Cin = z1.shape
    Dc = audio_features.shape[1]
    H = w2m.shape[0]
    Cout = b3.shape[1]

    max_rows = 8192
    nb = N
    if N * T > max_rows:
        for cand in range(min(N, max(1, max_rows // T)), 0, -1):
            if N % cand == 0 and cand % 8 == 0:
                nb = cand
                break
    grid = (N // nb,)
    R = nb * T
    n_chunks = max(1, R // 512)
    while nb % n_chunks != 0:
        n_chunks //= 2

    z1_flat = z1.reshape(N * T, Cin)
    af_b = audio_features.astype(jnp.bfloat16)
    wm_b = w_mlp_t.astype(jnp.bfloat16)
    w1_b = w1s.astype(jnp.bfloat16)
    w2_b = w2m.astype(jnp.bfloat16)
    w3_b = jnp.concatenate([w3s[0:H], w3s[H:2 * H], w3s[2 * H:3 * H]],
                           axis=1).astype(jnp.bfloat16)
    gw = jnp.tile(glow[:, 0:1], (nb, 1)).astype(jnp.bfloat16)
    gb = jnp.tile(glow[:, 1:2], (nb, 1)).astype(jnp.bfloat16)
    b12_b = b12.astype(jnp.bfloat16)

    plist = [wm_b, b_mlp, gw, gb, w1_b, w2_b, b12_b, w3_b, b3]

    in_specs = [
        pl.BlockSpec((nb, Dc), lambda g: (g, 0)),
        pl.BlockSpec((R, Cin), lambda g: (g, 0)),
    ] + [pl.BlockSpec(p.shape, lambda g: (0, 0)) for p in plist]

    out = pl.pallas_call(
        functools.partial(_fused_kernel, t_len=T, n_chunks=n_chunks),
        out_shape=jax.ShapeDtypeStruct((N * T, Cout), jnp.float32),
        grid=grid,
        in_specs=in_specs,
        out_specs=pl.BlockSpec((R, Cout), lambda g: (g, 0)),
        compiler_params=pltpu.CompilerParams(
            dimension_semantics=("parallel",),
            vmem_limit_bytes=64 * 1024 * 1024),
    )(af_b, z1_flat, *plist)
    return out.reshape(N, T, Cout)


# speedup vs baseline: 1.4306x; 1.2290x over previous
import functools

import jax
import jax.numpy as jnp
from jax import lax
from jax.experimental import pallas as pl
from jax.experimental.pallas import tpu as pltpu


def _fused_kernel(
    af_ref,
    z1_ref,
    wm_ref,
    bm_ref,
    glow_ref,
    w1_ref,
    w2_ref,
    b12_ref,
    w3_ref,
    b3_ref,
    o_ref,
    *,
    t_len,
    n_chunks,
):
    R = z1_ref.shape[0]
    cin = z1_ref.shape[1]
    hid = w2_ref.shape[0]
    cout = o_ref.shape[1]
    rc = R // n_chunks
    nbc = rc // t_len

    t_idx = lax.broadcasted_iota(jnp.int32, (rc, 1), 0) % t_len
    is_first = t_idx == 0
    is_last = t_idx == (t_len - 1)

    w1 = w1_ref[...].astype(jnp.bfloat16)
    w2 = w2_ref[...].astype(jnp.bfloat16)
    w3 = jnp.concatenate(
        [w3_ref[0:hid, :], w3_ref[hid:2 * hid, :], w3_ref[2 * hid:, :]],
        axis=-1).astype(jnp.bfloat16)
    b12 = b12_ref[...].astype(jnp.bfloat16)
    glow = glow_ref[...].astype(jnp.bfloat16)

    a_b = jnp.dot(af_ref[...].astype(jnp.bfloat16), wm_ref[...].astype(jnp.bfloat16),
                  preferred_element_type=jnp.float32) + bm_ref[...]
    a_b16 = a_b.astype(jnp.bfloat16)

    for c in range(n_chunks):
        rows = pl.ds(c * rc, rc)

        ab_c = a_b16[c * nbc:(c + 1) * nbc]
        af_glow = (ab_c[:, None, :] * glow[None, :, 0:1]
                   + glow[None, :, 1:2]).reshape(rc, cin)

        z = jnp.concatenate([z1_ref[rows, :].astype(jnp.bfloat16), af_glow],
                            axis=-1)
        z_prev = jnp.where(is_first, 0, pltpu.roll(z, 1, axis=0))
        z_next = jnp.where(is_last, 0, pltpu.roll(z, rc - 1, axis=0))
        zs = jnp.concatenate([z_prev, z, z_next], axis=-1)
        y = jnp.dot(zs, w1, preferred_element_type=jnp.float32)
        y = jnp.maximum(y.astype(jnp.bfloat16) + b12[0:1, :], 0)

        y = jnp.dot(y, w2, preferred_element_type=jnp.float32)
        y = jnp.maximum(y.astype(jnp.bfloat16) + b12[1:2, :], 0)

        p = jnp.dot(y, w3, preferred_element_type=jnp.float32)
        p_prev = pltpu.roll(p[:, :cout], 1, axis=0)
        p_next = pltpu.roll(p[:, 2 * cout:], rc - 1, axis=0)
        out = (p[:, cout:2 * cout]
               + jnp.where(is_first, 0.0, p_prev)
               + jnp.where(is_last, 0.0, p_next)
               + b3_ref[...])
        o_ref[rows, :] = out


@jax.jit
def kernel(z1, audio_features, w_mlp_t, b_mlp, glow, w1s, w2m, b12, w3s, b3):
    N, T, Cin = z1.shape
    Dc = audio_features.shape[1]
    Cout = b3.shape[1]

    max_rows = 8192
    nb = N
    if N * T > max_rows:
        for cand in range(min(N, max(1, max_rows // T)), 0, -1):
            if N % cand == 0 and cand % 8 == 0:
                nb = cand
                break
    grid = (N // nb,)
    R = nb * T
    n_chunks = max(1, R // 512)
    while nb % n_chunks != 0:
        n_chunks //= 2

    z1_flat = z1.reshape(N * T, Cin)

    plist = [w_mlp_t, b_mlp, glow, w1s, w2m, b12, w3s, b3]

    in_specs = [
        pl.BlockSpec((nb, Dc), lambda g: (g, 0)),
        pl.BlockSpec((R, Cin), lambda g: (g, 0)),
    ] + [pl.BlockSpec(p.shape, lambda g: (0, 0)) for p in plist]

    out = pl.pallas_call(
        functools.partial(_fused_kernel, t_len=T, n_chunks=n_chunks),
        out_shape=jax.ShapeDtypeStruct((N * T, Cout), jnp.float32),
        grid=grid,
        in_specs=in_specs,
        out_specs=pl.BlockSpec((R, Cout), lambda g: (g, 0)),
        compiler_params=pltpu.CompilerParams(
            dimension_semantics=("parallel",),
            vmem_limit_bytes=64 * 1024 * 1024),
    )(audio_features, z1_flat, *plist)
    return out.reshape(N, T, Cout)
